# baseline (speedup 1.0000x reference)
import functools

import jax
import jax.numpy as jnp
from jax import lax
from jax.experimental import pallas as pl
from jax.experimental.pallas import tpu as pltpu
from jax.experimental.pallas import tpu_sc as plsc

K = 32
R = 64
G = 16
CHK = 512
SLOTS = 16
CH = 4
NEG = -3.0e38


def _normalize_call(xpad):
    npad, d = xpad.shape
    br = 1024

    def body(x_ref, o_ref):
        xx = x_ref[...]
        nrm = jnp.sqrt(jnp.sum(xx * xx, axis=1, keepdims=True))
        o_ref[...] = xx / (nrm + 1e-8)

    return pl.pallas_call(
        body,
        grid=(npad // br,),
        in_specs=[pl.BlockSpec((br, d), lambda i: (i, 0))],
        out_specs=pl.BlockSpec((br, d), lambda i: (i, 0)),
        out_shape=jax.ShapeDtypeStruct((npad, d), jnp.float32),
    )(xpad)


def _topk_call(xn, xnt, n_real):
    npad, d = xn.shape
    ng = npad // G
    nch = npad // CHK

    def body(xr_ref, xnt_ref, w_ref, idx_ref):
        xr = xr_ref[...]
        panel = jnp.dot(xr, xnt_ref[...], preferred_element_type=jnp.float32)
        col2 = lax.broadcasted_iota(jnp.int32, (R, npad), 1)
        valid = col2 < n_real
        panel = jnp.where(valid, panel, NEG)

        gm = jnp.max(panel.reshape(R, ng, G), axis=2)
        cur = jnp.full((R, 1), jnp.inf, jnp.float32)
        for _ in range(K):
            cur = jnp.max(jnp.where(gm < cur, gm, -jnp.inf), axis=1,
                          keepdims=True)

        m = valid & (panel >= cur)

        m2d = m.astype(jnp.float32).reshape(R * nch, CHK)
        ia = lax.broadcasted_iota(jnp.int32, (CHK, CHK), 0)
        ib = lax.broadcasted_iota(jnp.int32, (CHK, CHK), 1)
        tri = (ia < ib).astype(jnp.float32)
        pos = jnp.dot(m2d, tri,
                      preferred_element_type=jnp.float32).reshape(R, nch, CHK)

        p3 = jnp.where(m, panel, NEG).reshape(R, nch, CHK)
        c3 = jnp.where(m, col2.astype(jnp.float32), -1.0).reshape(R, nch, CHK)
        cvals, cidxs = [], []
        for slot in range(SLOTS):
            sel = pos == float(slot)
            cvals.append(jnp.max(jnp.where(sel, p3, NEG), axis=2))
            cidxs.append(jnp.max(jnp.where(sel, c3, -1.0), axis=2))

        big = 1.0e9
        li = lax.broadcasted_iota(jnp.int32, (R, K), 1)
        vals = jnp.zeros((R, K), jnp.float32)
        idxs = jnp.zeros((R, K), jnp.float32)
        for k in range(K):
            mx = cvals[0]
            for s in range(1, SLOTS):
                mx = jnp.maximum(mx, cvals[s])
            mx = jnp.max(mx, axis=1, keepdims=True)
            pick = None
            for s in range(SLOTS):
                cand = jnp.min(jnp.where(cvals[s] == mx, cidxs[s], big),
                               axis=1, keepdims=True)
                pick = cand if pick is None else jnp.minimum(pick, cand)
            vals = jnp.where(li == k, mx, vals)
            idxs = jnp.where(li == k, pick, idxs)
            for s in range(SLOTS):
                hit = (cvals[s] == mx) & (cidxs[s] == pick)
                cvals[s] = jnp.where(hit, NEG, cvals[s])

        e = jnp.exp(vals - vals[:, 0:1])
        w_ref[...] = e / jnp.sum(e, axis=1, keepdims=True)
        idx_ref[...] = idxs.astype(jnp.int32)

    return pl.pallas_call(
        body,
        grid=(npad // R,),
        in_specs=[
            pl.BlockSpec((R, d), lambda i: (i, 0)),
            pl.BlockSpec((d, npad), lambda i: (0, 0)),
        ],
        out_specs=[
            pl.BlockSpec((R, K), lambda i: (i, 0)),
            pl.BlockSpec((R, K), lambda i: (i, 0)),
        ],
        out_shape=[
            jax.ShapeDtypeStruct((npad, K), jnp.float32),
            jax.ShapeDtypeStruct((npad, K), jnp.int32),
        ],
    )(xn, xnt)


def _dense_call(h, w, bias=None, leaky=False):
    npad, din = h.shape
    dout = w.shape[1]
    br = 1024

    def body(h_ref, w_ref, *rest):
        o_ref = rest[-1]
        acc = jnp.dot(h_ref[...], w_ref[...],
                      preferred_element_type=jnp.float32)
        if bias is not None:
            acc = acc + rest[0][...]
        if leaky:
            acc = jnp.where(acc >= 0, acc, acc * 0.01)
        o_ref[...] = acc

    in_specs = [
        pl.BlockSpec((br, din), lambda i: (i, 0)),
        pl.BlockSpec((din, dout), lambda i: (0, 0)),
    ]
    args = [h, w]
    if bias is not None:
        in_specs.append(pl.BlockSpec((1, dout), lambda i: (0, 0)))
        args.append(bias.reshape(1, dout))
    return pl.pallas_call(
        body,
        grid=(npad // br,),
        in_specs=in_specs,
        out_specs=pl.BlockSpec((br, dout), lambda i: (i, 0)),
        out_shape=jax.ShapeDtypeStruct((npad, dout), jnp.float32),
    )(*args)


def _prop_call(h_wide, idx_flat, w_flat, dh, leaky):
    npad = h_wide.shape[0]
    nw = 32
    rpw = npad // nw
    nchunk = rpw // CH
    mesh = plsc.VectorSubcoreMesh(core_axis_name="c", subcore_axis_name="s")

    @functools.partial(
        pl.kernel, mesh=mesh,
        out_type=jax.ShapeDtypeStruct((npad, 128), jnp.float32),
        scratch_types=[
            pltpu.VMEM((CH * K,), jnp.int32),
            pltpu.VMEM((CH * K,), jnp.float32),
            pltpu.VMEM((CH * K, 128), jnp.float32),
            pltpu.VMEM((CH, 128), jnp.float32),
            pltpu.SemaphoreType.DMA,
        ],
    )
    def sc_kernel(h_hbm, idx_hbm, w_hbm, out_hbm, idx_v, w_v, rows_v, out_v,
                  sem):
        wid = lax.axis_index("s") * 2 + lax.axis_index("c")
        base = wid * rpw

        zer = jnp.zeros((16,), jnp.float32)
        for r in range(CH):
            for j in range(dh // 16, 8):
                out_v[r, pl.ds(j * 16, 16)] = zer

        def chunk(ci, carry):
            rbase = base + ci * CH
            ebase = rbase * K
            pltpu.sync_copy(idx_hbm.at[pl.ds(ebase, CH * K)], idx_v)
            pltpu.sync_copy(w_hbm.at[pl.ds(ebase, CH * K)], w_v)
            pltpu.async_copy(h_hbm.at[idx_v], rows_v, sem).wait()
            for r in range(CH):
                wvecs = [w_v[pl.ds(r * K + 16 * b, 16)] for b in range(K // 16)]
                ws = [wvecs[k2 // 16][k2 % 16] for k2 in range(K)]
                for j in range(dh // 16):
                    acc = jnp.zeros((16,), jnp.float32)
                    for k2 in range(K):
                        acc = acc + rows_v[r * K + k2,
                                           pl.ds(j * 16, 16)] * ws[k2]
                    if leaky:
                        acc = jnp.where(acc >= 0, acc, acc * 0.01)
                    out_v[r, pl.ds(j * 16, 16)] = acc
            pltpu.sync_copy(out_v, out_hbm.at[pl.ds(rbase, CH)])
            return carry

        lax.fori_loop(0, nchunk, chunk, 0)

    return sc_kernel(h_wide, idx_flat, w_flat)


def _pad_w(w):
    r, c = w.shape
    return jnp.pad(w, ((0, 128 - r), (0, 128 - c)))


def kernel(x, W_enc1, W_enc2, W_dec1, W_dec2, W_pre, b_pre):
    n, d = x.shape
    npad = ((n + CHK - 1) // CHK) * CHK
    c = W_pre.shape[1]

    xpad = jnp.pad(x, ((0, npad - n), (0, 0)))
    xn = _normalize_call(xpad)
    xnt = xn.T

    wts, idx = _topk_call(xn, xnt, n)
    wflat = wts.reshape(-1)
    iflat = idx.reshape(-1)

    t1 = _dense_call(xpad, _pad_w(W_enc1))
    h1 = _prop_call(t1, iflat, wflat, 64, leaky=True)
    t2 = _dense_call(h1, _pad_w(W_enc2))
    h2 = _prop_call(t2, iflat, wflat, 32, leaky=True)
    p2 = _prop_call(h2, iflat, wflat, 32, leaky=False)
    d1 = _dense_call(p2, _pad_w(W_dec1), leaky=True)
    p3 = _prop_call(d1, iflat, wflat, 64, leaky=False)
    recon = _dense_call(p3, _pad_w(W_dec2))

    bp = jnp.pad(b_pre, (0, 128 - c))
    pre = _dense_call(h2, _pad_w(W_pre), bias=bp)

    return (h1[:n, :64], h2[:n, :32], recon[:n], pre[:n, :c])
